# Initial kernel scaffold; baseline (speedup 1.0000x reference)
import jax
import jax.numpy as jnp
from jax.experimental import pallas as pl


def kernel(x, nb, W, gamma, beta):
    raise NotImplementedError("write your pallas kernel here")


# speedup vs baseline: 213.0623x; 213.0623x over previous
import functools

import jax
import jax.numpy as jnp
from jax import lax
from jax.experimental import pallas as pl
from jax.experimental.pallas import tpu as pltpu
from jax.experimental.pallas import tpu_sc as plsc


def _sc_gather(idx3, x, nw, nch, k):
    total = nw * nch * k
    _, c = x.shape
    mesh = plsc.VectorSubcoreMesh(core_axis_name="c", subcore_axis_name="s")
    nc = mesh.num_cores

    @functools.partial(
        pl.kernel,
        out_type=jax.ShapeDtypeStruct((total, c), jnp.float32),
        mesh=mesh,
        scratch_types=[
            pltpu.VMEM((nch, k), jnp.int32),
            pltpu.VMEM((k, c), jnp.float32),
            pltpu.VMEM((k, c), jnp.float32),
            pltpu.SemaphoreType.DMA,
            pltpu.SemaphoreType.DMA,
        ],
    )
    def gather_kernel(idx_hbm, x_hbm, out_hbm, idx_v, bufa, bufb, sema, semb):
        wid = lax.axis_index("s") * nc + lax.axis_index("c")
        base = wid * (nch * k)
        pltpu.sync_copy(idx_hbm.at[wid], idx_v)

        def pair(j, carry):
            c0 = j * 2
            c1 = c0 + 1
            cpa = pltpu.async_copy(x_hbm.at[idx_v.at[c0]], bufa, sema)
            cpb = pltpu.async_copy(x_hbm.at[idx_v.at[c1]], bufb, semb)
            cpa.wait()
            pltpu.sync_copy(bufa, out_hbm.at[pl.ds(base + c0 * k, k)])
            cpb.wait()
            pltpu.sync_copy(bufb, out_hbm.at[pl.ds(base + c1 * k, k)])
            return carry

        lax.fori_loop(0, nch // 2, pair, 0)

    return gather_kernel(idx3, x)


def _mm_stats_body(x_ref, g_ref, w_ref, y_ref, s_ref):
    i = pl.program_id(0)
    g0 = g_ref[0]
    g1 = g_ref[1]
    g2 = g_ref[2]
    g3 = g_ref[3]
    feat = jnp.concatenate(
        [
            x_ref[...],
            jnp.minimum(g0, g1),
            jnp.maximum(g0, g1),
            jnp.minimum(g2, g3),
            jnp.maximum(g2, g3),
        ],
        axis=1,
    )
    y = jnp.dot(feat, w_ref[...], preferred_element_type=jnp.float32)
    y_ref[...] = y
    srow = jnp.sum(y, axis=0)[None]
    qrow = jnp.sum(y * y, axis=0)[None]
    blk = jnp.concatenate(
        [srow, qrow, jnp.zeros((6, y.shape[1]), jnp.float32)], axis=0
    )

    @pl.when(i == 0)
    def _():
        s_ref[...] = blk

    @pl.when(i != 0)
    def _():
        s_ref[...] += blk


def _norm_body(y_ref, s_ref, p_ref, o_ref, *, n_rows):
    inv_n = 1.0 / n_rows
    mean = s_ref[0] * inv_n
    var = s_ref[1] * inv_n - mean * mean
    inv = lax.rsqrt(var + 1e-5)
    scale = p_ref[0] * inv
    shift = p_ref[1] - mean * scale
    o_ref[...] = jnp.maximum(y_ref[...] * scale + shift, 0.0)


def kernel(x, nb, W, gamma, beta):
    e, c = x.shape
    c_out = W.shape[0]

    idx = jnp.clip(nb.astype(jnp.int32), 0, e - 1)
    idx_flat = idx.T.reshape(-1)

    nw = 32
    per_w = idx_flat.shape[0] // nw
    k = 80
    nch = per_w // k
    idx3 = idx_flat.reshape(nw, nch, k)

    g_flat = _sc_gather(idx3, x, nw, nch, k)
    g = g_flat.reshape(4, e, c)

    wt = W.T

    t = 1000
    grid = (e // t,)
    y, stats = pl.pallas_call(
        _mm_stats_body,
        grid=grid,
        in_specs=[
            pl.BlockSpec((t, c), lambda i: (i, 0)),
            pl.BlockSpec((4, t, c), lambda i: (0, i, 0)),
            pl.BlockSpec((5 * c, c_out), lambda i: (0, 0)),
        ],
        out_specs=[
            pl.BlockSpec((t, c_out), lambda i: (i, 0)),
            pl.BlockSpec((8, c_out), lambda i: (0, 0)),
        ],
        out_shape=[
            jax.ShapeDtypeStruct((e, c_out), jnp.float32),
            jax.ShapeDtypeStruct((8, c_out), jnp.float32),
        ],
    )(x, g, wt)

    params = jnp.concatenate(
        [gamma[None], beta[None], jnp.zeros((6, c_out), jnp.float32)], axis=0
    )

    out = pl.pallas_call(
        functools.partial(_norm_body, n_rows=e),
        grid=grid,
        in_specs=[
            pl.BlockSpec((t, c_out), lambda i: (i, 0)),
            pl.BlockSpec((8, c_out), lambda i: (0, 0)),
            pl.BlockSpec((8, c_out), lambda i: (0, 0)),
        ],
        out_specs=pl.BlockSpec((t, c_out), lambda i: (i, 0)),
        out_shape=jax.ShapeDtypeStruct((e, c_out), jnp.float32),
    )(y, stats, params)

    return out
